# baseline (speedup 1.0000x reference)
import functools

import jax
import jax.numpy as jnp
from jax import lax
from jax.experimental import pallas as pl
from jax.experimental.pallas import tpu as pltpu
from jax.experimental.pallas import tpu_sc as plsc

NC = 2
NS = 16
NW = NC * NS
L = 16

VOCAB = 1000000
BATCH = 4
CTX = 8192
EMBED = 64
S_PER_W = CTX // NW
SUB = 128
NSUB = S_PER_W // SUB
NCHUNK = BATCH * NSUB


def _sc_embed(x_flat, tok2, posT):
    mesh = plsc.VectorSubcoreMesh(core_axis_name="c", subcore_axis_name="s")

    @functools.partial(
        pl.kernel,
        out_type=jax.ShapeDtypeStruct((BATCH, EMBED, CTX), jnp.float32),
        mesh=mesh,
        scratch_types=[
            pltpu.VMEM((BATCH, S_PER_W), jnp.int32),
            pltpu.VMEM((NCHUNK, 1, SUB), jnp.int32),
            pltpu.VMEM((2, SUB, 128), jnp.float32),
            pltpu.VMEM((EMBED, S_PER_W), jnp.float32),
            pltpu.VMEM((2, EMBED, SUB), jnp.float32),
            pltpu.SemaphoreType.DMA,
            pltpu.SemaphoreType.DMA,
            pltpu.SemaphoreType.DMA,
            pltpu.SemaphoreType.DMA,
        ],
        compiler_params=pltpu.CompilerParams(needs_layout_passes=False),
    )
    def k(x_hbm, tok_hbm, pos_hbm, out_hbm, idx_v, rid_v, rows_v, pos_v,
          outb_v, gsem0, gsem1, psem, osem):
        wid = lax.axis_index("s") * NC + lax.axis_index("c")
        s_base = wid * S_PER_W

        pcopy = pltpu.async_copy(
            pos_hbm.at[:, pl.ds(s_base, S_PER_W)], pos_v, psem
        )

        icopies = [
            pltpu.sync_copy(
                x_hbm.at[pl.ds(b * CTX + s_base, S_PER_W)], idx_v.at[b]
            )
            for b in range(BATCH)
        ]

        for b in range(BATCH):
            for j in range(NSUB):
                c = b * NSUB + j
                def rid_body(g, _, b=b, j=j, c=c):
                    v = idx_v[b, pl.ds(j * SUB + g * L, L)]
                    rid = jax.lax.shift_left(
                        jax.lax.shift_right_logical(v, 12), 11
                    ) + jax.lax.bitwise_and(v, _R - 1)
                    rid_v[c, 0, pl.ds(g * L, L)] = rid
                    return 0
                lax.fori_loop(0, SUB // L, rid_body, 0)

        gsems = (gsem0, gsem1)

        def fire(c):
            buf = c % 2
            return pltpu.async_copy(
                tok_hbm.at[rid_v.at[c, 0]], rows_v.at[buf], gsems[buf]
            )

        g_prev = fire(0)
        pcopy.wait()

        row_iota = lax.broadcasted_iota(jnp.int32, (L,), 0)
        out_copies = []

        for c in range(NCHUNK):
            b, j = divmod(c, NSUB)
            g_next = fire(c + 1) if c + 1 < NCHUNK else None
            g_prev.wait()
            g_prev = g_next
            buf = c % 2

            if c >= 2:
                out_copies[c - 2].wait()

            def sel_body(sg, _, b=b, j=j, buf=buf):
                sl0 = sg * L
                iv = idx_v[b, pl.ds(j * SUB + sl0, L)]
                col_base = jax.lax.shift_left(
                    jax.lax.bitwise_and(
                        jax.lax.shift_right_logical(iv, 11), 1
                    ),
                    6,
                )
                rows = row_iota + sl0
                for e in range(EMBED):
                    g = plsc.load_gather(
                        rows_v.at[buf], [rows, col_base + e]
                    )
                    outb_v[buf, e, pl.ds(sl0, L)] = (
                        g + pos_v[e, pl.ds(j * SUB + sl0, L)]
                    )
                return 0

            lax.fori_loop(0, SUB // L, sel_body, 0)

            out_copies.append(
                pltpu.async_copy(
                    outb_v.at[buf],
                    out_hbm.at[b, :, pl.ds(s_base + j * SUB, SUB)],
                    osem,
                )
            )

        for c in (NCHUNK - 2, NCHUNK - 1):
            out_copies[c].wait()

    return k(x_flat, tok2, posT)


_R = 2048
_NSUPER = -(-VOCAB // (2 * _R))
_PACKED_ROWS = _NSUPER * _R


_NFULL = VOCAB // (2 * _R)
_TAIL = VOCAB - _NFULL * 2 * _R
_NTAIL = -(-_TAIL // 128)


def _pack_body(a1_ref, a2_ref, *rest):
    tail_refs, out_ref = rest[:_NTAIL], rest[_NTAIL]
    is_tail = pl.program_id(0) == _NSUPER - 1

    @pl.when(jnp.logical_not(is_tail))
    def _main():
        out_ref[...] = jnp.concatenate(
            [jnp.transpose(a1_ref[...]), jnp.transpose(a2_ref[...])], axis=1
        )

    @pl.when(is_tail)
    def _tail():
        lc = jnp.concatenate(
            [jnp.transpose(t[...]) for t in tail_refs]
            + [jnp.zeros((_R - 128 * _NTAIL, EMBED), jnp.float32)],
            axis=0,
        )
        out_ref[...] = jnp.concatenate([lc, lc], axis=1)


def _pack_table(tokT):
    last1 = _NFULL * 2 - 2
    last2 = _NFULL * 2 - 1
    tail0 = _NFULL * 2 * _R // 128
    tail_specs = [
        pl.BlockSpec((EMBED, 128), lambda i, t=t: (0, tail0 + t))
        for t in range(_NTAIL)
    ]
    return pl.pallas_call(
        _pack_body,
        grid=(_NSUPER,),
        in_specs=[
            pl.BlockSpec((EMBED, _R), lambda i: (0, jnp.minimum(2 * i, last1))),
            pl.BlockSpec(
                (EMBED, _R), lambda i: (0, jnp.minimum(2 * i + 1, last2))
            ),
        ]
        + tail_specs,
        out_specs=pl.BlockSpec((_R, 2 * EMBED), lambda i: (i, 0)),
        out_shape=jax.ShapeDtypeStruct((_PACKED_ROWS, 2 * EMBED), jnp.float32),
    )(tokT, tokT, *([tokT] * _NTAIL))


def kernel(x, tok_table, pos_table):
    x_flat = x.reshape(-1).astype(jnp.int32)
    tok2 = _pack_table(tok_table.T)
    posT = pos_table.T
    outT = _sc_embed(x_flat, tok2, posT)
    return outT.transpose(0, 2, 1)


# speedup vs baseline: 1.2054x; 1.2054x over previous
import functools

import jax
import jax.numpy as jnp
from jax import lax
from jax.experimental import pallas as pl
from jax.experimental.pallas import tpu as pltpu
from jax.experimental.pallas import tpu_sc as plsc

NC = 2
NS = 16
NW = NC * NS
L = 16

VOCAB = 1000000
BATCH = 4
CTX = 8192
EMBED = 64
S_PER_W = CTX // NW
SUB = 128
NSUB = S_PER_W // SUB
NCHUNK = BATCH * NSUB


def _sc_embed(x_flat, tok2, posT):
    mesh = plsc.VectorSubcoreMesh(core_axis_name="c", subcore_axis_name="s")

    @functools.partial(
        pl.kernel,
        out_type=jax.ShapeDtypeStruct((BATCH, EMBED, CTX), jnp.float32),
        mesh=mesh,
        scratch_types=[
            pltpu.VMEM((BATCH, S_PER_W), jnp.int32),
            pltpu.VMEM((NCHUNK, 1, SUB), jnp.int32),
            pltpu.VMEM((2, SUB, 128), jnp.float32),
            pltpu.VMEM((EMBED, S_PER_W), jnp.float32),
            pltpu.VMEM((2, EMBED, SUB), jnp.float32),
            pltpu.SemaphoreType.DMA,
            pltpu.SemaphoreType.DMA,
            pltpu.SemaphoreType.DMA,
            pltpu.SemaphoreType.DMA,
        ],
        compiler_params=pltpu.CompilerParams(needs_layout_passes=False),
    )
    def k(x_hbm, tok_hbm, pos_hbm, out_hbm, idx_v, rid_v, rows_v, pos_v,
          outb_v, gsem0, gsem1, psem, osem):
        wid = lax.axis_index("s") * NC + lax.axis_index("c")
        s_base = wid * S_PER_W

        pcopy = pltpu.async_copy(
            pos_hbm.at[:, pl.ds(s_base, S_PER_W)], pos_v, psem
        )

        icopies = [
            pltpu.sync_copy(
                x_hbm.at[pl.ds(b * CTX + s_base, S_PER_W)], idx_v.at[b]
            )
            for b in range(BATCH)
        ]

        for b in range(BATCH):
            for j in range(NSUB):
                c = b * NSUB + j
                def rid_body(g, _, b=b, j=j, c=c):
                    v = idx_v[b, pl.ds(j * SUB + g * L, L)]
                    rid = jax.lax.shift_left(
                        jax.lax.shift_right_logical(v, _RB + 1), _RB
                    ) + jax.lax.bitwise_and(v, _R - 1)
                    rid_v[c, 0, pl.ds(g * L, L)] = rid
                    return 0
                lax.fori_loop(0, SUB // L, rid_body, 0)

        gsems = (gsem0, gsem1)

        def fire(c):
            buf = c % 2
            return pltpu.async_copy(
                tok_hbm.at[rid_v.at[c, 0]], rows_v.at[buf], gsems[buf]
            )

        g_prev = fire(0)
        pcopy.wait()

        row_iota = lax.broadcasted_iota(jnp.int32, (L,), 0)
        out_copies = []

        for c in range(NCHUNK):
            b, j = divmod(c, NSUB)
            g_next = fire(c + 1) if c + 1 < NCHUNK else None
            g_prev.wait()
            g_prev = g_next
            buf = c % 2

            if c >= 2:
                out_copies[c - 2].wait()

            def sel_body(sg, _, b=b, j=j, buf=buf):
                sl0 = sg * L
                iv = idx_v[b, pl.ds(j * SUB + sl0, L)]
                col_base = jax.lax.shift_left(
                    jax.lax.bitwise_and(
                        jax.lax.shift_right_logical(iv, _RB), 1
                    ),
                    6,
                )
                rows = row_iota + sl0
                for e in range(EMBED):
                    g = plsc.load_gather(
                        rows_v.at[buf], [rows, col_base + e]
                    )
                    outb_v[buf, e, pl.ds(sl0, L)] = (
                        g + pos_v[e, pl.ds(j * SUB + sl0, L)]
                    )
                return 0

            lax.fori_loop(0, SUB // L, sel_body, 0)

            out_copies.append(
                pltpu.async_copy(
                    outb_v.at[buf],
                    out_hbm.at[b, :, pl.ds(s_base + j * SUB, SUB)],
                    osem,
                )
            )

        for c in (NCHUNK - 2, NCHUNK - 1):
            out_copies[c].wait()

    return k(x_flat, tok2, posT)


_R = 4096
_RB = _R.bit_length() - 1
_NSUPER = -(-VOCAB // (2 * _R))
_PACKED_ROWS = _NSUPER * _R


_NFULL = VOCAB // (2 * _R)
_TAIL = VOCAB - _NFULL * 2 * _R
_NTAIL = -(-_TAIL // 128)


def _pack_body(a1_ref, a2_ref, *rest):
    tail_refs, out_ref = rest[:_NTAIL], rest[_NTAIL]
    is_tail = pl.program_id(0) == _NSUPER - 1

    @pl.when(jnp.logical_not(is_tail))
    def _main():
        out_ref[...] = jnp.concatenate(
            [jnp.transpose(a1_ref[...]), jnp.transpose(a2_ref[...])], axis=1
        )

    @pl.when(is_tail)
    def _tail():
        lc = jnp.concatenate(
            [jnp.transpose(t[...]) for t in tail_refs]
            + [jnp.zeros((_R - 128 * _NTAIL, EMBED), jnp.float32)],
            axis=0,
        )
        out_ref[...] = jnp.concatenate([lc, lc], axis=1)


def _pack_table(tokT):
    last1 = _NFULL * 2 - 2
    last2 = _NFULL * 2 - 1
    tail0 = _NFULL * 2 * _R // 128
    tail_specs = [
        pl.BlockSpec((EMBED, 128), lambda i, t=t: (0, tail0 + t))
        for t in range(_NTAIL)
    ]
    return pl.pallas_call(
        _pack_body,
        grid=(_NSUPER,),
        in_specs=[
            pl.BlockSpec((EMBED, _R), lambda i: (0, jnp.minimum(2 * i, last1))),
            pl.BlockSpec(
                (EMBED, _R), lambda i: (0, jnp.minimum(2 * i + 1, last2))
            ),
        ]
        + tail_specs,
        out_specs=pl.BlockSpec((_R, 2 * EMBED), lambda i: (i, 0)),
        out_shape=jax.ShapeDtypeStruct((_PACKED_ROWS, 2 * EMBED), jnp.float32),
    )(tokT, tokT, *([tokT] * _NTAIL))


def kernel(x, tok_table, pos_table):
    x_flat = x.reshape(-1).astype(jnp.int32)
    tok2 = _pack_table(tok_table.T)
    posT = pos_table.T
    outT = _sc_embed(x_flat, tok2, posT)
    return outT.transpose(0, 2, 1)


# speedup vs baseline: 1.3420x; 1.1133x over previous
import functools

import jax
import jax.numpy as jnp
from jax import lax
from jax.experimental import pallas as pl
from jax.experimental.pallas import tpu as pltpu
from jax.experimental.pallas import tpu_sc as plsc

NC = 2
NS = 16
NW = NC * NS
L = 16

VOCAB = 1000000
BATCH = 4
CTX = 8192
EMBED = 64
S_PER_W = CTX // NW
SUB = 128
NSUB = S_PER_W // SUB
NCHUNK = BATCH * NSUB


def _sc_embed(x_flat, tok2, posT):
    mesh = plsc.VectorSubcoreMesh(core_axis_name="c", subcore_axis_name="s")

    @functools.partial(
        pl.kernel,
        out_type=jax.ShapeDtypeStruct((BATCH, EMBED, CTX), jnp.float32),
        mesh=mesh,
        scratch_types=[
            pltpu.VMEM((BATCH, S_PER_W), jnp.int32),
            pltpu.VMEM((NCHUNK, 1, SUB), jnp.int32),
            pltpu.VMEM((2, SUB, 128), jnp.float32),
            pltpu.VMEM((EMBED, S_PER_W), jnp.float32),
            pltpu.VMEM((2, EMBED, SUB), jnp.float32),
            pltpu.SemaphoreType.DMA,
            pltpu.SemaphoreType.DMA,
            pltpu.SemaphoreType.DMA,
            pltpu.SemaphoreType.DMA,
        ],
        compiler_params=pltpu.CompilerParams(needs_layout_passes=False),
    )
    def k(x_hbm, tok_hbm, pos_hbm, out_hbm, idx_v, rid_v, rows_v, pos_v,
          outb_v, gsem0, gsem1, psem, osem):
        wid = lax.axis_index("s") * NC + lax.axis_index("c")
        s_base = wid * S_PER_W

        pcopy = pltpu.async_copy(
            pos_hbm.at[:, pl.ds(s_base, S_PER_W)], pos_v, psem
        )

        icopies = [
            pltpu.sync_copy(
                x_hbm.at[pl.ds(b * CTX + s_base, S_PER_W)], idx_v.at[b]
            )
            for b in range(BATCH)
        ]

        for b in range(BATCH):
            for j in range(NSUB):
                c = b * NSUB + j
                def rid_body(g, _, b=b, j=j, c=c):
                    v = idx_v[b, pl.ds(j * SUB + g * L, L)]
                    rid = jax.lax.shift_left(
                        jax.lax.shift_right_logical(v, _RB + 1), _RB
                    ) + jax.lax.bitwise_and(v, _R - 1)
                    rid_v[c, 0, pl.ds(g * L, L)] = rid
                    return 0
                lax.fori_loop(0, SUB // L, rid_body, 0)

        gsems = (gsem0, gsem1)

        def fire(c):
            buf = c % 2
            return pltpu.async_copy(
                tok_hbm.at[rid_v.at[c, 0]], rows_v.at[buf], gsems[buf]
            )

        g_prev = fire(0)
        pcopy.wait()

        row_iota = lax.broadcasted_iota(jnp.int32, (L,), 0)
        out_copies = []

        for c in range(NCHUNK):
            b, j = divmod(c, NSUB)
            g_next = fire(c + 1) if c + 1 < NCHUNK else None
            g_prev.wait()
            g_prev = g_next
            buf = c % 2

            if c >= 2:
                out_copies[c - 2].wait()

            def sel_body(sg, _, b=b, j=j, buf=buf):
                sl0 = sg * L
                iv = idx_v[b, pl.ds(j * SUB + sl0, L)]
                col_base = jax.lax.shift_left(
                    jax.lax.bitwise_and(
                        jax.lax.shift_right_logical(iv, _RB), 1
                    ),
                    6,
                )
                rows = row_iota + sl0
                for e in range(EMBED):
                    g = plsc.load_gather(
                        rows_v.at[buf], [rows, col_base + e]
                    )
                    outb_v[buf, e, pl.ds(sl0, L)] = (
                        g + pos_v[e, pl.ds(j * SUB + sl0, L)]
                    )
                return 0

            lax.fori_loop(0, SUB // L, sel_body, 0)

            out_copies.append(
                pltpu.async_copy(
                    outb_v.at[buf],
                    out_hbm.at[b, :, pl.ds(s_base + j * SUB, SUB)],
                    osem,
                )
            )

        for c in (NCHUNK - 2, NCHUNK - 1):
            out_copies[c].wait()

    return k(x_flat, tok2, posT)


_R = 8192
_RB = _R.bit_length() - 1
_NSUPER = -(-VOCAB // (2 * _R))
_PACKED_ROWS = _NSUPER * _R


_NFULL = VOCAB // (2 * _R)
_TAIL = VOCAB - _NFULL * 2 * _R
_NTAIL = -(-_TAIL // 128)


def _pack_body(a1_ref, a2_ref, *rest):
    tail_refs, out_ref = rest[:_NTAIL], rest[_NTAIL]
    is_tail = pl.program_id(0) == _NSUPER - 1

    @pl.when(jnp.logical_not(is_tail))
    def _main():
        out_ref[...] = jnp.concatenate(
            [jnp.transpose(a1_ref[...]), jnp.transpose(a2_ref[...])], axis=1
        )

    @pl.when(is_tail)
    def _tail():
        lc = jnp.concatenate(
            [jnp.transpose(t[...]) for t in tail_refs]
            + [jnp.zeros((_R - 128 * _NTAIL, EMBED), jnp.float32)],
            axis=0,
        )
        out_ref[...] = jnp.concatenate([lc, lc], axis=1)


def _pack_table(tokT):
    last1 = _NFULL * 2 - 2
    last2 = _NFULL * 2 - 1
    tail0 = _NFULL * 2 * _R // 128
    tail_specs = [
        pl.BlockSpec((EMBED, 128), lambda i, t=t: (0, tail0 + t))
        for t in range(_NTAIL)
    ]
    return pl.pallas_call(
        _pack_body,
        grid=(_NSUPER,),
        in_specs=[
            pl.BlockSpec((EMBED, _R), lambda i: (0, jnp.minimum(2 * i, last1))),
            pl.BlockSpec(
                (EMBED, _R), lambda i: (0, jnp.minimum(2 * i + 1, last2))
            ),
        ]
        + tail_specs,
        out_specs=pl.BlockSpec((_R, 2 * EMBED), lambda i: (i, 0)),
        out_shape=jax.ShapeDtypeStruct((_PACKED_ROWS, 2 * EMBED), jnp.float32),
    )(tokT, tokT, *([tokT] * _NTAIL))


def kernel(x, tok_table, pos_table):
    x_flat = x.reshape(-1).astype(jnp.int32)
    tok2 = _pack_table(tok_table.T)
    posT = pos_table.T
    outT = _sc_embed(x_flat, tok2, posT)
    return outT.transpose(0, 2, 1)


# speedup vs baseline: 1.4011x; 1.0440x over previous
import functools

import jax
import jax.numpy as jnp
from jax import lax
from jax.experimental import pallas as pl
from jax.experimental.pallas import tpu as pltpu
from jax.experimental.pallas import tpu_sc as plsc

NC = 2
NS = 16
NW = NC * NS
L = 16

VOCAB = 1000000
BATCH = 4
CTX = 8192
EMBED = 64
S_PER_W = CTX // NW
SUB = 128
NSUB = S_PER_W // SUB
NCHUNK = BATCH * NSUB


def _sc_embed(x_flat, tok2, posT):
    mesh = plsc.VectorSubcoreMesh(core_axis_name="c", subcore_axis_name="s")

    @functools.partial(
        pl.kernel,
        out_type=jax.ShapeDtypeStruct((BATCH, EMBED, CTX), jnp.float32),
        mesh=mesh,
        scratch_types=[
            pltpu.VMEM((BATCH, S_PER_W), jnp.int32),
            pltpu.VMEM((NCHUNK, 1, SUB), jnp.int32),
            pltpu.VMEM((2, SUB, 128), jnp.float32),
            pltpu.VMEM((EMBED, S_PER_W), jnp.float32),
            pltpu.VMEM((2, EMBED, SUB), jnp.float32),
            pltpu.SemaphoreType.DMA,
            pltpu.SemaphoreType.DMA,
            pltpu.SemaphoreType.DMA,
            pltpu.SemaphoreType.DMA,
        ],
        compiler_params=pltpu.CompilerParams(needs_layout_passes=False),
    )
    def k(x_hbm, tok_hbm, pos_hbm, out_hbm, idx_v, rid_v, rows_v, pos_v,
          outb_v, gsem0, gsem1, psem, osem):
        wid = lax.axis_index("s") * NC + lax.axis_index("c")
        s_base = wid * S_PER_W

        pcopy = pltpu.async_copy(
            pos_hbm.at[:, pl.ds(s_base, S_PER_W)], pos_v, psem
        )

        icopies = [
            pltpu.sync_copy(
                x_hbm.at[pl.ds(b * CTX + s_base, S_PER_W)], idx_v.at[b]
            )
            for b in range(BATCH)
        ]

        for b in range(BATCH):
            for j in range(NSUB):
                c = b * NSUB + j
                def rid_body(g, _, b=b, j=j, c=c):
                    v = idx_v[b, pl.ds(j * SUB + g * L, L)]
                    rid = jax.lax.shift_left(
                        jax.lax.shift_right_logical(v, _RB + 1), _RB
                    ) + jax.lax.bitwise_and(v, _R - 1)
                    rid_v[c, 0, pl.ds(g * L, L)] = rid
                    return 0
                lax.fori_loop(0, SUB // L, rid_body, 0)

        gsems = (gsem0, gsem1)

        def fire(c):
            buf = c % 2
            return pltpu.async_copy(
                tok_hbm.at[rid_v.at[c, 0]], rows_v.at[buf], gsems[buf]
            )

        g_prev = fire(0)
        pcopy.wait()

        row_iota = lax.broadcasted_iota(jnp.int32, (L,), 0)
        out_copies = []

        for c in range(NCHUNK):
            b, j = divmod(c, NSUB)
            g_next = fire(c + 1) if c + 1 < NCHUNK else None
            g_prev.wait()
            g_prev = g_next
            buf = c % 2

            if c >= 2:
                out_copies[c - 2].wait()

            def sel_body(sg, _, b=b, j=j, buf=buf):
                sl0 = sg * L
                iv = idx_v[b, pl.ds(j * SUB + sl0, L)]
                col_base = jax.lax.shift_left(
                    jax.lax.bitwise_and(
                        jax.lax.shift_right_logical(iv, _RB), 1
                    ),
                    6,
                )
                rows = row_iota + sl0
                for e in range(EMBED):
                    g = plsc.load_gather(
                        rows_v.at[buf], [rows, col_base + e]
                    )
                    outb_v[buf, e, pl.ds(sl0, L)] = (
                        g + pos_v[e, pl.ds(j * SUB + sl0, L)]
                    )
                return 0

            lax.fori_loop(0, SUB // L, sel_body, 0)

            out_copies.append(
                pltpu.async_copy(
                    outb_v.at[buf],
                    out_hbm.at[b, :, pl.ds(s_base + j * SUB, SUB)],
                    osem,
                )
            )

        for c in (NCHUNK - 2, NCHUNK - 1):
            out_copies[c].wait()

    return k(x_flat, tok2, posT)


_R = 16384
_RB = _R.bit_length() - 1
_NSUPER = -(-VOCAB // (2 * _R))
_PACKED_ROWS = _NSUPER * _R


def _pack_body(a1_ref, a2_ref, out_ref):
    out_ref[...] = jnp.concatenate(
        [jnp.transpose(a1_ref[...]), jnp.transpose(a2_ref[...])], axis=1
    )


def _pack_table(tokT):
    last = -(-VOCAB // _R) - 1
    return pl.pallas_call(
        _pack_body,
        grid=(_NSUPER,),
        in_specs=[
            pl.BlockSpec((EMBED, _R), lambda i: (0, jnp.minimum(2 * i, last))),
            pl.BlockSpec(
                (EMBED, _R), lambda i: (0, jnp.minimum(2 * i + 1, last))
            ),
        ],
        out_specs=pl.BlockSpec((_R, 2 * EMBED), lambda i: (i, 0)),
        out_shape=jax.ShapeDtypeStruct((_PACKED_ROWS, 2 * EMBED), jnp.float32),
    )(tokT, tokT)


def kernel(x, tok_table, pos_table):
    x_flat = x.reshape(-1).astype(jnp.int32)
    tok2 = _pack_table(tok_table.T)
    posT = pos_table.T
    outT = _sc_embed(x_flat, tok2, posT)
    return outT.transpose(0, 2, 1)


# speedup vs baseline: 1.5160x; 1.0820x over previous
import functools

import jax
import jax.numpy as jnp
from jax import lax
from jax.experimental import pallas as pl
from jax.experimental.pallas import tpu as pltpu
from jax.experimental.pallas import tpu_sc as plsc

NC = 2
NS = 16
NW = NC * NS
L = 16

VOCAB = 1000000
BATCH = 4
CTX = 8192
EMBED = 64
S_PER_W = CTX // NW
SUB = 128
NSUB = S_PER_W // SUB
NCHUNK = BATCH * NSUB


def _sc_gather(x_flat, tok2):
    mesh = plsc.VectorSubcoreMesh(core_axis_name="c", subcore_axis_name="s")

    @functools.partial(
        pl.kernel,
        out_type=jax.ShapeDtypeStruct((BATCH * CTX, 2 * EMBED), jnp.float32),
        mesh=mesh,
        scratch_types=[
            pltpu.VMEM((BATCH, S_PER_W), jnp.int32),
            pltpu.VMEM((NCHUNK, 1, SUB), jnp.int32),
            pltpu.VMEM((3, SUB, 128), jnp.float32),
            pltpu.SemaphoreType.DMA,
            pltpu.SemaphoreType.DMA,
            pltpu.SemaphoreType.DMA,
            pltpu.SemaphoreType.DMA,
        ],
        compiler_params=pltpu.CompilerParams(needs_layout_passes=False),
    )
    def k(x_hbm, tok_hbm, out_hbm, idx_v, rid_v, rows_v, gsem0, gsem1,
          gsem2, osem):
        wid = lax.axis_index("s") * NC + lax.axis_index("c")
        s_base = wid * S_PER_W

        for b in range(BATCH):
            pltpu.sync_copy(
                x_hbm.at[pl.ds(b * CTX + s_base, S_PER_W)], idx_v.at[b]
            )

        for b in range(BATCH):
            for j in range(NSUB):
                c = b * NSUB + j
                def rid_body(g, _, b=b, j=j, c=c):
                    v = idx_v[b, pl.ds(j * SUB + g * L, L)]
                    rid = jax.lax.shift_left(
                        jax.lax.shift_right_logical(v, _RB + 1), _RB
                    ) + jax.lax.bitwise_and(v, _R - 1)
                    rid_v[c, 0, pl.ds(g * L, L)] = rid
                    return 0
                lax.fori_loop(0, SUB // L, rid_body, 0)

        gsems = (gsem0, gsem1, gsem2)

        def fire(c):
            buf = c % 3
            return pltpu.async_copy(
                tok_hbm.at[rid_v.at[c, 0]], rows_v.at[buf], gsems[buf]
            )

        g_prev = fire(0)
        out_copies = []
        for c in range(NCHUNK):
            b, j = divmod(c, NSUB)
            g_next = None
            if c + 1 < NCHUNK:
                if c >= 2:
                    out_copies[c - 2].wait()
                g_next = fire(c + 1)
            g_prev.wait()
            g_prev = g_next
            out_copies.append(
                pltpu.async_copy(
                    rows_v.at[c % 3],
                    out_hbm.at[pl.ds(b * CTX + s_base + j * SUB, SUB)],
                    osem,
                )
            )

        for c in (NCHUNK - 3, NCHUNK - 2, NCHUNK - 1):
            out_copies[c].wait()

    return k(x_flat, tok2)


_SB = 1024


def _epi_body(raw_ref, x_ref, pos_ref, out_ref):
    p = jax.lax.bitwise_and(
        jax.lax.shift_right_logical(x_ref[pl.ds(pl.program_id(0), 1), :], _RB),
        1,
    )
    rawT = jnp.transpose(raw_ref[...])
    sel = jnp.where(p == 1, rawT[EMBED:, :], rawT[:EMBED, :])
    out_ref[0] = sel + pos_ref[...]


def _epilogue(raw, x, posT):
    nsb = CTX // _SB
    return pl.pallas_call(
        _epi_body,
        grid=(BATCH, nsb),
        in_specs=[
            pl.BlockSpec((_SB, 2 * EMBED), lambda b, s: (b * nsb + s, 0)),
            pl.BlockSpec((BATCH, _SB), lambda b, s: (0, s)),
            pl.BlockSpec((EMBED, _SB), lambda b, s: (0, s)),
        ],
        out_specs=pl.BlockSpec((1, EMBED, _SB), lambda b, s: (b, 0, s)),
        out_shape=jax.ShapeDtypeStruct((BATCH, EMBED, CTX), jnp.float32),
    )(raw, x, posT)


_R = 16384
_RB = _R.bit_length() - 1
_NSUPER = -(-VOCAB // (2 * _R))
_PACKED_ROWS = _NSUPER * _R


def _pack_body(a1_ref, a2_ref, out_ref):
    out_ref[...] = jnp.concatenate(
        [jnp.transpose(a1_ref[...]), jnp.transpose(a2_ref[...])], axis=1
    )


def _pack_table(tokT):
    last = -(-VOCAB // _R) - 1
    return pl.pallas_call(
        _pack_body,
        grid=(_NSUPER,),
        in_specs=[
            pl.BlockSpec((EMBED, _R), lambda i: (0, jnp.minimum(2 * i, last))),
            pl.BlockSpec(
                (EMBED, _R), lambda i: (0, jnp.minimum(2 * i + 1, last))
            ),
        ],
        out_specs=pl.BlockSpec((_R, 2 * EMBED), lambda i: (i, 0)),
        out_shape=jax.ShapeDtypeStruct((_PACKED_ROWS, 2 * EMBED), jnp.float32),
    )(tokT, tokT)


def kernel(x, tok_table, pos_table):
    x = x.astype(jnp.int32)
    tok2 = _pack_table(tok_table.T)
    raw = _sc_gather(x.reshape(-1), tok2)
    outT = _epilogue(raw, x, pos_table.T)
    return outT.transpose(0, 2, 1)
